# baseline (speedup 1.0000x reference)
import jax
import jax.numpy as jnp
from jax.experimental import pallas as pl
from jax.experimental.pallas import tpu as pltpu

N = 10000
BLK = 400


def _critic_body(x_ref, w1_ref, b1_ref, w2_ref, b2_ref, o_ref):
    h = jnp.maximum(
        jax.lax.dot_general(x_ref[...], w1_ref[...], (((1,), (0,)), ((), ())),
                            preferred_element_type=jnp.float32) + b1_ref[...],
        0.0)
    o_ref[...] = jax.lax.dot_general(h, w2_ref[...], (((1,), (0,)), ((), ())),
                                     preferred_element_type=jnp.float32) + b2_ref[...]


def _critic(x3, cW1, cb1, cW2, cb2):
    grid = (N // BLK,)
    out = pl.pallas_call(
        _critic_body,
        grid=grid,
        in_specs=[
            pl.BlockSpec((BLK, 256), lambda i: (i, 0)),
            pl.BlockSpec((256, 256), lambda i: (0, 0)),
            pl.BlockSpec((256,), lambda i: (0,)),
            pl.BlockSpec((256, 128), lambda i: (0, 0)),
            pl.BlockSpec((128,), lambda i: (0,)),
        ],
        out_specs=pl.BlockSpec((BLK, 128), lambda i: (i, 0)),
        out_shape=jax.ShapeDtypeStruct((N, 128), jnp.float32),
    )(x3, cW1, cb1, jnp.pad(cW2, ((0, 0), (0, 127))), jnp.pad(cb2, (0, 127)))
    return out[:, 0]


def kernel(x_goal, x_obs, x_task, x_value, edge_attr_ot, edge_index_go,
           edge_index_ot, edge_index_tv, sW1, sb1, sW2, sb2, tWe, tbe,
           tW1, tb1, tW2, tb2, gWl, gbl, gWr, gbr, ga, gb,
           cW1, cb1, cW2, cb2):
    src, dst = edge_index_go[0], edge_index_go[1]
    aggr = jax.ops.segment_sum(x_goal[src], dst, num_segments=N)
    h = jnp.maximum((x_obs + aggr) @ sW1 + sb1, 0.0)
    x1 = h @ sW2 + sb2
    src, dst = edge_index_ot[0], edge_index_ot[1]
    msg = jnp.maximum(x1[src] + (edge_attr_ot @ tWe + tbe), 0.0)
    aggr = jax.ops.segment_sum(msg, dst, num_segments=N)
    h = jnp.maximum((x_task + aggr) @ tW1 + tb1, 0.0)
    x2 = h @ tW2 + tb2
    src, dst = edge_index_tv[0], edge_index_tv[1]
    xl = x2 @ gWl + gbl
    xr = x_value @ gWr + gbr
    feat = xl[src] + xr[dst]
    feat = jnp.where(feat > 0, feat, 0.2 * feat)
    logits = feat @ ga
    m = jax.ops.segment_max(logits, dst, num_segments=N)
    ex = jnp.exp(logits - m[dst])
    denom = jax.ops.segment_sum(ex, dst, num_segments=N)
    alpha = ex / denom[dst]
    x3 = jax.ops.segment_sum(alpha[:, None] * xl[src], dst, num_segments=N) + gb
    return _critic(x3, cW1, cb1, cW2, cb2)


# speedup vs baseline: 1.6281x; 1.6281x over previous
import jax
import jax.numpy as jnp
from jax import lax
from jax.experimental import pallas as pl
from jax.experimental.pallas import tpu as pltpu
from jax.experimental.pallas import tpu_sc as plsc

N = 10000
E = 160000
D = 256
H = 128
RW = 80
NR = E // RW
NB8 = NR // 8
NS = 16
BLK = 400

_mesh = plsc.VectorSubcoreMesh(core_axis_name="c", subcore_axis_name="s")
_f32 = jnp.float32


def _nblocks(s):
    return jnp.where(s < 10, 16, 15)


N2 = 10240


def _slab1d(mk_src, mk_dst, s):
    pltpu.sync_copy(mk_src(s * 640, 640), mk_dst(s * 640, 640))


def _slab_copy(mk_src, mk_dst, s, add=False):
    pl.when(s < 15)(lambda: pltpu.sync_copy(
        mk_src(s * 640, 640), mk_dst(s * 640, 640), add=add))
    pl.when(s == 15)(lambda: pltpu.sync_copy(
        mk_src(9600, 400), mk_dst(9600, 400), add=add))


def _segsum_body(xlo, xhi, srcR, dstR, zNH, out, idxs_v, idxd_v, rows_v,
                 acc_sh, sem):
    c = lax.axis_index("c")
    s = lax.axis_index("s")
    _slab_copy(lambda o, n: zNH.at[pl.ds(o, n)],
               lambda o, n: acc_sh.at[pl.ds(o, n)], s)
    plsc.subcore_barrier()

    def run(xsrc):
        def chunk(k, _):
            r = k * 128 + s * 8
            pltpu.sync_copy(srcR.at[pl.ds(r, 8)], idxs_v)
            pltpu.sync_copy(dstR.at[pl.ds(r, 8)], idxd_v)
            for hh in range(2):
                cps = [pltpu.async_copy(xsrc.at[idxs_v.at[hh * 4 + j]],
                                        rows_v.at[j], sem) for j in range(4)]
                for cp in cps:
                    cp.wait()

                def scat(j, _):
                    pltpu.sync_copy(rows_v.at[j],
                                    acc_sh.at[idxd_v.at[hh * 4 + j]],
                                    add=True)
                    return 0
                lax.fori_loop(0, 4, scat, 0)
            return 0
        lax.fori_loop(0, _nblocks(s), chunk, 0)

    pl.when(c == 0)(lambda: run(xlo))
    pl.when(c == 1)(lambda: run(xhi))
    plsc.subcore_barrier()
    _slab_copy(lambda o, n: acc_sh.at[pl.ds(o, n)],
               lambda o, n: out.at[c, pl.ds(o, n)], s)


_segsum = pl.kernel(
    _segsum_body,
    out_type=jax.ShapeDtypeStruct((2, N, H), _f32),
    mesh=_mesh,
    compiler_params=pltpu.CompilerParams(needs_layout_passes=False),
    scratch_types=[
        pltpu.VMEM((8, RW), jnp.int32),
        pltpu.VMEM((8, RW), jnp.int32),
        pltpu.VMEM((4, RW, H), _f32),
        pltpu.VMEM_SHARED((N, H), _f32),
        pltpu.SemaphoreType.DMA,
    ],
)


def _gine_body(xlo, xhi, srcR, dstR, eaA, eaB, twe_lo, twe_hi, tbe_lo,
               tbe_hi, zNH, out, idxs_v, idxd_v, rows_v, ea0_v, ea1_v, w_v,
               b_v, acc_sh, sem):
    c = lax.axis_index("c")
    s = lax.axis_index("s")
    _slab_copy(lambda o, n: zNH.at[pl.ds(o, n)],
               lambda o, n: acc_sh.at[pl.ds(o, n)], s)

    def _ld(tw, tb):
        def f():
            pltpu.sync_copy(tw, w_v)
            pltpu.sync_copy(tb, b_v)
        return f
    pl.when(c == 0)(_ld(twe_lo, tbe_lo))
    pl.when(c == 1)(_ld(twe_hi, tbe_hi))
    plsc.subcore_barrier()

    def run(xsrc):
        def chunk(k, _):
            r = k * 128 + s * 8
            pltpu.sync_copy(srcR.at[pl.ds(r, 8)], idxs_v)
            pltpu.sync_copy(dstR.at[pl.ds(r, 8)], idxd_v)
            pltpu.sync_copy(eaA.at[pl.ds(r, 8)], ea0_v)
            pltpu.sync_copy(eaB.at[pl.ds(r, 8)], ea1_v)
            for hh in range(2):
                cps = [pltpu.async_copy(xsrc.at[idxs_v.at[hh * 4 + j]],
                                        rows_v.at[j], sem) for j in range(4)]
                for cp in cps:
                    cp.wait()

                def row(j, _):
                    def mblk(m, _):
                        av = ea0_v[hh * 4 + j, pl.ds(m * 16, 16)]
                        bv = ea1_v[hh * 4 + j, pl.ds(m * 16, 16)]
                        for i in range(16):
                            e = m * 16 + i
                            a = av[i]
                            b = bv[i]
                            for q in range(8):
                                sl = pl.ds(q * 16, 16)
                                v = (rows_v[j, e, sl] + a * w_v[0, sl]
                                     + b * w_v[1, sl] + b_v[sl])
                                rows_v[j, e, sl] = jnp.maximum(v, 0.0)
                        return 0
                    lax.fori_loop(0, RW // 16, mblk, 0)
                    return 0
                lax.fori_loop(0, 4, row, 0)

                def scat(j, _):
                    pltpu.sync_copy(rows_v.at[j],
                                    acc_sh.at[idxd_v.at[hh * 4 + j]],
                                    add=True)
                    return 0
                lax.fori_loop(0, 4, scat, 0)
            return 0
        lax.fori_loop(0, _nblocks(s), chunk, 0)

    pl.when(c == 0)(lambda: run(xlo))
    pl.when(c == 1)(lambda: run(xhi))
    plsc.subcore_barrier()
    _slab_copy(lambda o, n: acc_sh.at[pl.ds(o, n)],
               lambda o, n: out.at[c, pl.ds(o, n)], s)


_gine = pl.kernel(
    _gine_body,
    out_type=jax.ShapeDtypeStruct((2, N, H), _f32),
    mesh=_mesh,
    compiler_params=pltpu.CompilerParams(needs_layout_passes=False),
    scratch_types=[
        pltpu.VMEM((8, RW), jnp.int32),
        pltpu.VMEM((8, RW), jnp.int32),
        pltpu.VMEM((4, RW, H), _f32),
        pltpu.VMEM((8, RW), _f32),
        pltpu.VMEM((8, RW), _f32),
        pltpu.VMEM((2, H), _f32),
        pltpu.VMEM((H,), _f32),
        pltpu.VMEM_SHARED((N, H), _f32),
        pltpu.SemaphoreType.DMA,
    ],
)


def _logits_body(xl_lo, xl_hi, xr_lo, xr_hi, srcR, dstR, ga_lo, ga_hi,
                 lgA, lgB, idxs_v, idxd_v, rl_v, rr_v, g_v, lg_v, sem):
    c = lax.axis_index("c")
    s = lax.axis_index("s")
    lidx = lax.iota(jnp.int32, 16)

    def run(xl, xr, gah, lgout):
        pltpu.sync_copy(gah, g_v)

        def chunk(k, _):
            r = k * 128 + s * 8
            pltpu.sync_copy(srcR.at[pl.ds(r, 8)], idxs_v)
            pltpu.sync_copy(dstR.at[pl.ds(r, 8)], idxd_v)
            for hh in range(2):
                cps = [pltpu.async_copy(xl.at[idxs_v.at[hh * 4 + j]],
                                        rl_v.at[j], sem) for j in range(4)]
                cps += [pltpu.async_copy(xr.at[idxd_v.at[hh * 4 + j]],
                                         rr_v.at[j], sem) for j in range(4)]
                for cp in cps:
                    cp.wait()

                def row(j, _):
                    jv = jnp.full((16,), j, jnp.int32)

                    def mblk(m, _):
                        eids = m * 16 + lidx
                        acc = jnp.zeros((16,), _f32)
                        for cb in range(8):
                            gw = g_v[pl.ds(cb * 16, 16)]
                            for t in range(16):
                                cv = jnp.full((16,), cb * 16 + t, jnp.int32)
                                v = (plsc.load_gather(rl_v, [jv, eids, cv])
                                     + plsc.load_gather(rr_v, [jv, eids, cv]))
                                v = jnp.where(v > 0, v, 0.2 * v)
                                acc = acc + gw[t] * v
                        lg_v[hh * 4 + j, pl.ds(m * 16, 16)] = acc
                        return 0
                    lax.fori_loop(0, RW // 16, mblk, 0)
                    return 0
                lax.fori_loop(0, 4, row, 0)
            pltpu.sync_copy(lg_v, lgout.at[pl.ds(r, 8)])
            return 0
        lax.fori_loop(0, _nblocks(s), chunk, 0)

    pl.when(c == 0)(lambda: run(xl_lo, xr_lo, ga_lo, lgA))
    pl.when(c == 1)(lambda: run(xl_hi, xr_hi, ga_hi, lgB))


_gat_logits = pl.kernel(
    _logits_body,
    out_type=(jax.ShapeDtypeStruct((NR, RW), _f32),
              jax.ShapeDtypeStruct((NR, RW), _f32)),
    mesh=_mesh,
    compiler_params=pltpu.CompilerParams(needs_layout_passes=False),
    scratch_types=[
        pltpu.VMEM((8, RW), jnp.int32),
        pltpu.VMEM((8, RW), jnp.int32),
        pltpu.VMEM((4, RW, H), _f32),
        pltpu.VMEM((4, RW, H), _f32),
        pltpu.VMEM((H,), _f32),
        pltpu.VMEM((8, RW), _f32),
        pltpu.SemaphoreType.DMA,
    ],
)


def _combine_body(xl_lo, xl_hi, srcR, dstR, lgA, lgB, zNH, zN,
                  x3, den0, den1, idxs_v, idxd_v, rows_v, la_v, lb_v, al_v,
                  dn_v, mx_v, mxr_v, acc_sh, den_sh, mx_sh, sem):
    c = lax.axis_index("c")
    s = lax.axis_index("s")
    _slab_copy(lambda o, n: zNH.at[pl.ds(o, n)],
               lambda o, n: acc_sh.at[pl.ds(o, n)], s)
    _slab1d(lambda o, n: zN.at[pl.ds(o, n)],
            lambda o, n: den_sh.at[pl.ds(o, n)], s)

    def chunk0(k, mv):
        r = k * 128 + s * 8
        pltpu.sync_copy(lgA.at[pl.ds(r, 8)], la_v)
        pltpu.sync_copy(lgB.at[pl.ds(r, 8)], lb_v)

        def row(j, mvj):
            def mblk(m, mvm):
                sl = pl.ds(m * 16, 16)
                return jnp.maximum(mvm, la_v[j, sl] + lb_v[j, sl])
            return lax.fori_loop(0, RW // 16, mblk, mvj)
        return lax.fori_loop(0, 8, row, mv)
    mv = lax.fori_loop(0, _nblocks(s), chunk0,
                       jnp.full((16,), -jnp.inf, _f32))
    mx_v[pl.ds(0, 16)] = mv
    pltpu.sync_copy(mx_v, mx_sh.at[s * 8])
    plsc.subcore_barrier()
    pltpu.sync_copy(mx_sh, mxr_v)
    mv = jnp.full((16,), -jnp.inf, _f32)
    for t in range(NS):
        mv = jnp.maximum(mv, mxr_v[t * 8, pl.ds(0, 16)])
    gmx = jnp.max(mv)
    plsc.subcore_barrier()

    def chunk1(k, _):
        r = k * 128 + s * 8
        pltpu.sync_copy(lgA.at[pl.ds(r, 8)], la_v)
        pltpu.sync_copy(lgB.at[pl.ds(r, 8)], lb_v)
        pltpu.sync_copy(dstR.at[pl.ds(r, 8)], idxd_v)

        def row(j, _):
            def mblk(m, _):
                sl = pl.ds(m * 16, 16)
                al_v[j, sl] = jnp.exp(la_v[j, sl] + lb_v[j, sl] - gmx)
                return 0
            lax.fori_loop(0, RW // 16, mblk, 0)
            return 0
        lax.fori_loop(0, 8, row, 0)

        def scat(j, _):
            pltpu.sync_copy(al_v.at[j], den_sh.at[idxd_v.at[j]], add=True)
            return 0
        lax.fori_loop(0, 8, scat, 0)
        return 0
    lax.fori_loop(0, _nblocks(s), chunk1, 0)
    plsc.subcore_barrier()

    def den_out(dref):
        _slab1d(lambda o, n: den_sh.at[pl.ds(o, n)],
                lambda o, n: dref.at[pl.ds(o, n)], s)

    pl.when(c == 0)(lambda: den_out(den0))
    pl.when(c == 1)(lambda: den_out(den1))
    plsc.subcore_barrier()

    def run(xsrc, dref):
        def chunk2(k, _):
            r = k * 128 + s * 8
            pltpu.sync_copy(srcR.at[pl.ds(r, 8)], idxs_v)
            pltpu.sync_copy(dstR.at[pl.ds(r, 8)], idxd_v)
            pltpu.sync_copy(lgA.at[pl.ds(r, 8)], la_v)
            pltpu.sync_copy(lgB.at[pl.ds(r, 8)], lb_v)
            cps = [pltpu.async_copy(dref.at[idxd_v.at[j]], dn_v.at[j], sem)
                   for j in range(8)]
            for cp in cps:
                cp.wait()

            def rowa(j, _):
                def mblk(m, _):
                    sl = pl.ds(m * 16, 16)
                    al_v[j, sl] = (jnp.exp(la_v[j, sl] + lb_v[j, sl] - gmx)
                                   / dn_v[j, sl])
                    return 0
                lax.fori_loop(0, RW // 16, mblk, 0)
                return 0
            lax.fori_loop(0, 8, rowa, 0)
            for hh in range(4):
                cps = [pltpu.async_copy(xsrc.at[idxs_v.at[hh * 2 + j]],
                                        rows_v.at[j], sem) for j in range(2)]
                for cp in cps:
                    cp.wait()

                def row(j, _):
                    def mblk2(m, _):
                        aw = al_v[hh * 2 + j, pl.ds(m * 16, 16)]
                        for i in range(16):
                            e = m * 16 + i
                            a = aw[i]
                            for q in range(8):
                                sl = pl.ds(q * 16, 16)
                                rows_v[j, e, sl] = rows_v[j, e, sl] * a
                        return 0
                    lax.fori_loop(0, RW // 16, mblk2, 0)
                    return 0
                lax.fori_loop(0, 2, row, 0)

                def scat(j, _):
                    pltpu.sync_copy(rows_v.at[j],
                                    acc_sh.at[idxd_v.at[hh * 2 + j]],
                                    add=True)
                    return 0
                lax.fori_loop(0, 2, scat, 0)
            return 0
        lax.fori_loop(0, _nblocks(s), chunk2, 0)

    pl.when(c == 0)(lambda: run(xl_lo, den0))
    pl.when(c == 1)(lambda: run(xl_hi, den1))
    plsc.subcore_barrier()
    _slab_copy(lambda o, n: acc_sh.at[pl.ds(o, n)],
               lambda o, n: x3.at[c, pl.ds(o, n)], s)


_gat_combine = pl.kernel(
    _combine_body,
    out_type=(jax.ShapeDtypeStruct((2, N, H), _f32),
              jax.ShapeDtypeStruct((N2,), _f32),
              jax.ShapeDtypeStruct((N2,), _f32)),
    mesh=_mesh,
    compiler_params=pltpu.CompilerParams(needs_layout_passes=False),
    scratch_types=[
        pltpu.VMEM((8, RW), jnp.int32),
        pltpu.VMEM((8, RW), jnp.int32),
        pltpu.VMEM((2, RW, H), _f32),
        pltpu.VMEM((8, RW), _f32),
        pltpu.VMEM((8, RW), _f32),
        pltpu.VMEM((8, RW), _f32),
        pltpu.VMEM((8, RW), _f32),
        pltpu.VMEM((16,), _f32),
        pltpu.VMEM((NS * 8, 16), _f32),
        pltpu.VMEM_SHARED((N, H), _f32),
        pltpu.VMEM_SHARED((N2,), _f32),
        pltpu.VMEM_SHARED((NS * 8, 16), _f32),
        pltpu.SemaphoreType.DMA,
    ],
)


def _dot(a, b):
    return jax.lax.dot_general(a, b, (((1,), (0,)), ((), ())),
                               preferred_element_type=_f32)


def _stage_b_body(xo_ref, ag_ref, w1, b1, w2, b2, xv_ref, wr, br,
                  o1_ref, o2_ref):
    x = xo_ref[...] + jnp.concatenate([ag_ref[0], ag_ref[1]], axis=1)
    h = jnp.maximum(_dot(x, w1[...]) + b1[...], 0.0)
    x1 = _dot(h, w2[...]) + b2[...]
    o1_ref[0] = x1[:, :H]
    o1_ref[1] = x1[:, H:]
    xr = _dot(xv_ref[...], wr[...]) + br[...]
    o2_ref[0] = xr[:, :H]
    o2_ref[1] = xr[:, H:]


def _stage_b(x_obs, aggr1, sW1, sb1, sW2, sb2, x_value, gWr, gbr):
    grid = (N // BLK,)
    return pl.pallas_call(
        _stage_b_body,
        grid=grid,
        in_specs=[
            pl.BlockSpec((BLK, D), lambda i: (i, 0)),
            pl.BlockSpec((2, BLK, H), lambda i: (0, i, 0)),
            pl.BlockSpec((D, D), lambda i: (0, 0)),
            pl.BlockSpec((D,), lambda i: (0,)),
            pl.BlockSpec((D, D), lambda i: (0, 0)),
            pl.BlockSpec((D,), lambda i: (0,)),
            pl.BlockSpec((BLK, D), lambda i: (i, 0)),
            pl.BlockSpec((D, D), lambda i: (0, 0)),
            pl.BlockSpec((D,), lambda i: (0,)),
        ],
        out_specs=[pl.BlockSpec((2, BLK, H), lambda i: (0, i, 0)),
                   pl.BlockSpec((2, BLK, H), lambda i: (0, i, 0))],
        out_shape=[jax.ShapeDtypeStruct((2, N, H), _f32),
                   jax.ShapeDtypeStruct((2, N, H), _f32)],
    )(x_obs, aggr1, sW1, sb1, sW2, sb2, x_value, gWr, gbr)


def _stage_d_body(xt_ref, ag_ref, w1, b1, w2, b2, wl, bl, o_ref):
    x = xt_ref[...] + jnp.concatenate([ag_ref[0], ag_ref[1]], axis=1)
    h = jnp.maximum(_dot(x, w1[...]) + b1[...], 0.0)
    x2 = _dot(h, w2[...]) + b2[...]
    xl = _dot(x2, wl[...]) + bl[...]
    o_ref[0] = xl[:, :H]
    o_ref[1] = xl[:, H:]


def _stage_d(x_task, aggr2, tW1, tb1, tW2, tb2, gWl, gbl):
    grid = (N // BLK,)
    return pl.pallas_call(
        _stage_d_body,
        grid=grid,
        in_specs=[
            pl.BlockSpec((BLK, D), lambda i: (i, 0)),
            pl.BlockSpec((2, BLK, H), lambda i: (0, i, 0)),
            pl.BlockSpec((D, D), lambda i: (0, 0)),
            pl.BlockSpec((D,), lambda i: (0,)),
            pl.BlockSpec((D, D), lambda i: (0, 0)),
            pl.BlockSpec((D,), lambda i: (0,)),
            pl.BlockSpec((D, D), lambda i: (0, 0)),
            pl.BlockSpec((D,), lambda i: (0,)),
        ],
        out_specs=pl.BlockSpec((2, BLK, H), lambda i: (0, i, 0)),
        out_shape=jax.ShapeDtypeStruct((2, N, H), _f32),
    )(x_task, aggr2, tW1, tb1, tW2, tb2, gWl, gbl)


def _critic_body(x3_ref, gb, w1, b1, w2, b2, o_ref):
    x = jnp.concatenate([x3_ref[0], x3_ref[1]], axis=1) + gb[...]
    h = jnp.maximum(_dot(x, w1[...]) + b1[...], 0.0)
    o_ref[...] = _dot(h, w2[...]) + b2[...]


def _critic(x3, gb, cW1, cb1, cW2, cb2):
    grid = (N // BLK,)
    out = pl.pallas_call(
        _critic_body,
        grid=grid,
        in_specs=[
            pl.BlockSpec((2, BLK, H), lambda i: (0, i, 0)),
            pl.BlockSpec((D,), lambda i: (0,)),
            pl.BlockSpec((D, D), lambda i: (0, 0)),
            pl.BlockSpec((D,), lambda i: (0,)),
            pl.BlockSpec((D, H), lambda i: (0, 0)),
            pl.BlockSpec((H,), lambda i: (0,)),
        ],
        out_specs=pl.BlockSpec((BLK, H), lambda i: (i, 0)),
        out_shape=jax.ShapeDtypeStruct((N, H), _f32),
    )(x3, gb, cW1, cb1, jnp.pad(cW2, ((0, 0), (0, H - 1))),
      jnp.pad(cb2, (0, H - 1)))
    return out[:, 0]


def kernel(x_goal, x_obs, x_task, x_value, edge_attr_ot, edge_index_go,
           edge_index_ot, edge_index_tv, sW1, sb1, sW2, sb2, tWe, tbe,
           tW1, tb1, tW2, tb2, gWl, gbl, gWr, gbr, ga, gb,
           cW1, cb1, cW2, cb2):
    sgo = edge_index_go[0].reshape(NR, RW)
    dgo = edge_index_go[1].reshape(NR, RW)
    sot = edge_index_ot[0].reshape(NR, RW)
    dot_ = edge_index_ot[1].reshape(NR, RW)
    stv = edge_index_tv[0].reshape(NR, RW)
    dtv = edge_index_tv[1].reshape(NR, RW)
    eaA = edge_attr_ot[:, 0].reshape(NR, RW)
    eaB = edge_attr_ot[:, 1].reshape(NR, RW)
    zNH = jnp.zeros((N, H), _f32)
    zN = jnp.zeros((N2,), _f32)

    aggr1 = _segsum(x_goal[:, :H], x_goal[:, H:], sgo, dgo, zNH)
    x1, xr = _stage_b(x_obs, aggr1, sW1, sb1, sW2, sb2, x_value, gWr, gbr)
    aggr2 = _gine(x1[0], x1[1], sot, dot_, eaA, eaB,
                  tWe[:, :H], tWe[:, H:], tbe[:H], tbe[H:], zNH)
    xl = _stage_d(x_task, aggr2, tW1, tb1, tW2, tb2, gWl, gbl)
    lgA, lgB = _gat_logits(xl[0], xl[1], xr[0], xr[1], stv, dtv,
                           ga[:H], ga[H:])
    x3, _d0, _d1 = _gat_combine(xl[0], xl[1], stv, dtv, lgA, lgB, zNH, zN)
    return _critic(x3, gb, cW1, cb1, cW2, cb2)
